# baseline (speedup 1.0000x reference)
import jax
import jax.numpy as jnp
from jax import lax
from jax.experimental import pallas as pl
from jax.experimental.pallas import tpu as pltpu

_B, _Q, _NT, _C = 8, 256, 32, 32
_WCE, _WBB, _WGI = 1.0, 5.0, 2.0


def _loss_body(big_ref, lbl_ref, tb_ref, tc_ref, out_ref):
    bigT = jnp.concatenate(
        [jnp.transpose(big_ref[b]).reshape(1, 62, _Q) for b in range(_B)], axis=0
    )
    x3 = bigT[:, 0:_C, :]
    pb3 = bigT[:, _C : _C + 6, :]
    ce_pos = jnp.sum(jnp.maximum(x3, 0.0) + jnp.log(1.0 + jnp.exp(-jnp.abs(x3))))

    lbl3 = lbl_ref[...]
    y3 = jnp.zeros((_B, _NT, _Q), jnp.float32)
    for g in range(4):
        sub = jnp.clip(lbl3 - 8 * g, 0, 7)
        subB = jnp.broadcast_to(sub, (_B, _NT, _Q))
        part = jnp.take_along_axis(x3[:, 8 * g : 8 * g + 8, :], subB, axis=1)
        y3 = y3 + jnp.where((lbl3 >= 8 * g) & (lbl3 < 8 * g + 8), part, 0.0)

    tb3 = tb_ref[...]
    cbb3 = jnp.zeros((_B, _NT, _Q), jnp.float32)
    for dd in range(6):
        cbb3 = cbb3 + jnp.abs(pb3[:, dd : dd + 1, :] - tb3[:, :, dd : dd + 1])
    cost3 = -(1.0 / (1.0 + jnp.exp(-y3))) + cbb3

    costT = [jnp.transpose(cost3[:, j, :]) for j in range(_NT)]
    q_iota_s = lax.broadcasted_iota(jnp.int32, (_Q, 1), 0)
    usedT = jnp.zeros((_Q, _B), jnp.float32)
    rows = []
    for j in range(_NT):
        cv = jnp.where(usedT > 0.5, jnp.inf, costT[j])
        m = jnp.min(cv, axis=0, keepdims=True)
        idx = jnp.min(jnp.where(cv == m, q_iota_s, _Q), axis=0, keepdims=True)
        ohqT = jnp.where(q_iota_s == idx, 1.0, 0.0)
        usedT = jnp.maximum(usedT, ohqT)
        rows.append(jnp.transpose(ohqT).reshape(_B, 1, _Q))

    st3 = jnp.concatenate(rows, axis=1)
    xz = jnp.sum(st3 * y3)
    bbox = jnp.sum(st3 * cbb3)

    giou_s = jnp.float32(0.0)
    for b in range(_B):
        pcT = bigT[b, _C + 6 :, :]
        mins, maxs = [], []
        for dd in range(3):
            lo = pcT[dd : dd + 1, :]
            hi = pcT[dd : dd + 1, :]
            for k in range(1, 8):
                ck = pcT[3 * k + dd : 3 * k + dd + 1, :]
                lo = jnp.minimum(lo, ck)
                hi = jnp.maximum(hi, ck)
            mins.append(lo)
            maxs.append(hi)
        sm6 = jnp.concatenate(mins + maxs, axis=0)
        mm = lax.dot_general(
            sm6,
            st3[b],
            (((1,), (1,)), ((), ())),
            precision=lax.Precision.HIGHEST,
            preferred_element_type=jnp.float32,
        )
        tcT = jnp.transpose(tc_ref[b])
        inter = jnp.float32(1.0)
        vol_s = jnp.float32(1.0)
        vol_t = jnp.float32(1.0)
        enc = jnp.float32(1.0)
        for dd in range(3):
            smn = mm[dd : dd + 1, :]
            smx = mm[3 + dd : 4 + dd, :]
            tmn = tcT[dd : dd + 1, :]
            tmx = tcT[dd : dd + 1, :]
            for k in range(1, 8):
                ck = tcT[3 * k + dd : 3 * k + dd + 1, :]
                tmn = jnp.minimum(tmn, ck)
                tmx = jnp.maximum(tmx, ck)
            inter = inter * jnp.maximum(jnp.minimum(smx, tmx) - jnp.maximum(smn, tmn), 0.0)
            vol_s = vol_s * (smx - smn)
            vol_t = vol_t * (tmx - tmn)
            enc = enc * (jnp.maximum(smx, tmx) - jnp.minimum(smn, tmn))
        union = vol_s + vol_t - inter
        g = inter / (union + 1e-7) - (enc - union) / (enc + 1e-7)
        giou_s = giou_s + jnp.sum(g)

    ce = (ce_pos - xz) / (_B * _Q * _C)
    bb = bbox / (_B * _NT * 6)
    gi = 1.0 - giou_s / (_B * _NT)
    out_ref[0] = ce * _WCE + bb * _WBB + gi * _WGI
    out_ref[1] = ce
    out_ref[2] = bb
    out_ref[3] = gi


def kernel(pred_logits, pred_boxes, pred_corners, tgt_labels, tgt_boxes, tgt_corners):
    big = jnp.concatenate(
        [pred_logits, pred_boxes, pred_corners.reshape(_B, _Q, 24)], axis=-1
    )
    lbl = tgt_labels.astype(jnp.int32).reshape(_B, _NT, 1)
    tc24 = tgt_corners.reshape(_B, _NT, 24)
    out = pl.pallas_call(
        _loss_body,
        out_shape=jax.ShapeDtypeStruct((4,), jnp.float32),
        out_specs=pl.BlockSpec(memory_space=pltpu.SMEM),
    )(big, lbl, tgt_boxes, tc24)
    return (out[0], out[1], out[2], out[3])


# speedup vs baseline: 1.3229x; 1.3229x over previous
import jax
import jax.numpy as jnp
from jax import lax
from jax.experimental import pallas as pl
from jax.experimental.pallas import tpu as pltpu

_B, _Q, _NT, _C = 8, 256, 32, 32
_WCE, _WBB, _WGI = 1.0, 5.0, 2.0


def _loss_body(xT_ref, pbT_ref, pcT_ref, lbl_ref, tb_ref, tcT_ref, out_ref):
    x3 = xT_ref[...]
    ce_pos = jnp.sum(jnp.maximum(x3, 0.0) + jnp.log(1.0 + jnp.exp(-jnp.abs(x3))))

    lbl3 = lbl_ref[...]
    y3 = jnp.zeros((_B, _NT, _Q), jnp.float32)
    for g in range(4):
        sub = jnp.clip(lbl3 - 8 * g, 0, 7)
        subB = jnp.broadcast_to(sub, (_B, _NT, _Q))
        part = jnp.take_along_axis(x3[:, 8 * g : 8 * g + 8, :], subB, axis=1)
        y3 = y3 + jnp.where((lbl3 >= 8 * g) & (lbl3 < 8 * g + 8), part, 0.0)

    pb3 = pbT_ref[...]
    tb3 = tb_ref[...]
    cbb3 = jnp.zeros((_B, _NT, _Q), jnp.float32)
    for dd in range(6):
        cbb3 = cbb3 + jnp.abs(pb3[:, dd : dd + 1, :] - tb3[:, :, dd : dd + 1])
    cost3 = -(1.0 / (1.0 + jnp.exp(-y3))) + cbb3

    costT = [jnp.transpose(cost3[:, j, :]) for j in range(_NT)]
    q_iota_s = lax.broadcasted_iota(jnp.int32, (_Q, 1), 0)
    usedT = jnp.zeros((_Q, _B), jnp.float32)
    rows = []
    for j in range(_NT):
        cv = jnp.where(usedT > 0.5, jnp.inf, costT[j])
        m = jnp.min(cv, axis=0, keepdims=True)
        idx = jnp.min(jnp.where(cv == m, q_iota_s, _Q), axis=0, keepdims=True)
        ohqT = jnp.where(q_iota_s == idx, 1.0, 0.0)
        usedT = jnp.maximum(usedT, ohqT)
        rows.append(jnp.transpose(ohqT).reshape(_B, 1, _Q))

    st3 = jnp.concatenate(rows, axis=1)
    xz = jnp.sum(st3 * y3)
    bbox = jnp.sum(st3 * cbb3)

    smin = pcT_ref[:, 0]
    smax = pcT_ref[:, 0]
    for k in range(1, 8):
        ck = pcT_ref[:, k]
        smin = jnp.minimum(smin, ck)
        smax = jnp.maximum(smax, ck)

    giou_s = jnp.float32(0.0)
    for b in range(_B):
        sm6 = jnp.concatenate([smin[b], smax[b]], axis=0)
        mm = lax.dot_general(
            sm6,
            st3[b],
            (((1,), (1,)), ((), ())),
            precision=lax.Precision.HIGHEST,
            preferred_element_type=jnp.float32,
        )
        inter = jnp.float32(1.0)
        vol_s = jnp.float32(1.0)
        vol_t = jnp.float32(1.0)
        enc = jnp.float32(1.0)
        for dd in range(3):
            smn = mm[dd : dd + 1, :]
            smx = mm[3 + dd : 4 + dd, :]
            tmn = tcT_ref[b, dd, 0:1]
            tmx = tcT_ref[b, dd, 0:1]
            for k in range(1, 8):
                ck = tcT_ref[b, dd, k : k + 1]
                tmn = jnp.minimum(tmn, ck)
                tmx = jnp.maximum(tmx, ck)
            inter = inter * jnp.maximum(jnp.minimum(smx, tmx) - jnp.maximum(smn, tmn), 0.0)
            vol_s = vol_s * (smx - smn)
            vol_t = vol_t * (tmx - tmn)
            enc = enc * (jnp.maximum(smx, tmx) - jnp.minimum(smn, tmn))
        union = vol_s + vol_t - inter
        g = inter / (union + 1e-7) - (enc - union) / (enc + 1e-7)
        giou_s = giou_s + jnp.sum(g)

    ce = (ce_pos - xz) / (_B * _Q * _C)
    bb = bbox / (_B * _NT * 6)
    gi = 1.0 - giou_s / (_B * _NT)
    out_ref[0] = ce * _WCE + bb * _WBB + gi * _WGI
    out_ref[1] = ce
    out_ref[2] = bb
    out_ref[3] = gi


def kernel(pred_logits, pred_boxes, pred_corners, tgt_labels, tgt_boxes, tgt_corners):
    xT = jnp.transpose(pred_logits, (0, 2, 1))
    pbT = jnp.transpose(pred_boxes, (0, 2, 1))
    pcT = jnp.transpose(pred_corners, (0, 2, 3, 1))
    lbl = tgt_labels.astype(jnp.int32).reshape(_B, _NT, 1)
    tcT = jnp.transpose(tgt_corners, (0, 3, 2, 1))
    out = pl.pallas_call(
        _loss_body,
        out_shape=jax.ShapeDtypeStruct((4,), jnp.float32),
        out_specs=pl.BlockSpec(memory_space=pltpu.SMEM),
    )(xT, pbT, pcT, lbl, tgt_boxes, tcT)
    return (out[0], out[1], out[2], out[3])
